# Initial kernel scaffold; baseline (speedup 1.0000x reference)
import jax
import jax.numpy as jnp
from jax.experimental import pallas as pl


def kernel(x_start, t, alpha_bars):
    raise NotImplementedError("write your pallas kernel here")


# speedup vs baseline: 1.0279x; 1.0279x over previous
import numpy as np
import jax
import jax.numpy as jnp
from jax import lax
from jax.experimental import pallas as pl
from jax.experimental.pallas import tpu as pltpu

VOCAB = 100000
ROWS, COLS = 128, 4096
TIMESTEPS = 1000
MOD31 = (1 << 31) % VOCAB

_ROTS = ((13, 15, 26, 6), (17, 29, 16, 24))


def _np_threefry(k0, k1, x0, x1):
    with np.errstate(over="ignore"):
        k0, k1 = np.uint32(k0), np.uint32(k1)
        x0, x1 = np.uint32(x0), np.uint32(x1)
        ks = (k0, k1, np.uint32(k0 ^ k1 ^ np.uint32(0x1BD11BDA)))
        x0 = x0 + ks[0]
        x1 = x1 + ks[1]
        for i in range(5):
            for r in _ROTS[i % 2]:
                x0 = x0 + x1
                x1 = (x1 << np.uint32(r)) | (x1 >> np.uint32(32 - r))
                x1 = x1 ^ x0
            x0 = x0 + ks[(i + 1) % 3]
            x1 = x1 + ks[(i + 2) % 3] + np.uint32(i + 1)
        return x0, x1


def _np_split(k):
    a0, b0 = _np_threefry(k[0], k[1], 0, 0)
    a1, b1 = _np_threefry(k[0], k[1], 0, 1)
    return (a0, b0), (a1, b1)


_KN, _KB = _np_split((np.uint32(0), np.uint32(42)))
_K1, _K2 = _np_split(_KN)


def _u32(v):
    return np.uint32(v)


def _tf_bits(k0, k1, x1_in):
    ks0 = _u32(k0)
    ks1 = _u32(k1)
    ks2 = _u32(int(k0) ^ int(k1) ^ 0x1BD11BDA)
    ks = (ks0, ks1, ks2)
    x0 = jnp.full(x1_in.shape, ks0, jnp.uint32)
    x1 = x1_in + ks1
    for i in range(5):
        for r in _ROTS[i % 2]:
            x0 = x0 + x1
            x1 = (x1 << _u32(r)) | (x1 >> _u32(32 - r))
            x1 = x1 ^ x0
        x0 = x0 + ks[(i + 1) % 3]
        x1 = x1 + _u32(int(ks[(i + 2) % 3]) + i + 1)
    return x0 ^ x1


def _umod_vocab(bits_u32):
    x31 = (bits_u32 & _u32(0x7FFFFFFF)).astype(jnp.int32)
    q = (x31.astype(jnp.float32) * np.float32(1.0 / VOCAB)).astype(jnp.int32)
    r = x31 - q * VOCAB
    r = jnp.where(r < 0, r + VOCAB, r)
    r = jnp.where(r >= VOCAB, r - VOCAB, r)
    r = r + jnp.where(bits_u32 >= _u32(0x80000000), MOD31, 0)
    return jnp.where(r >= VOCAB, r - VOCAB, r)


def _body(t_ref, ab_ref, x_ref, o_ref):
    t = t_ref[:]
    steps = lax.broadcasted_iota(jnp.int32, (ROWS, TIMESTEPS), 1)
    ab = ab_ref[:]
    a_row = jnp.sum(jnp.where(t == steps, ab, 0.0), axis=1, keepdims=True)
    thr = jnp.ceil(a_row * np.float32(1 << 23)).astype(jnp.int32)

    row = lax.broadcasted_iota(jnp.uint32, (ROWS, COLS), 0)
    col = lax.broadcasted_iota(jnp.uint32, (ROWS, COLS), 1)
    idx = row * _u32(COLS) + col

    noise = _umod_vocab(_tf_bits(_K2[0], _K2[1], idx))
    ubits = _tf_bits(_KB[0], _KB[1], idx)
    mant = (ubits >> _u32(9)).astype(jnp.int32)
    keep = mant < thr
    o_ref[:] = jnp.where(keep, x_ref[:], noise)


@jax.jit
def kernel(x_start, t, alpha_bars):
    x_start = x_start.astype(jnp.int32)
    t2 = t.astype(jnp.int32).reshape(ROWS, 1)
    ab2 = alpha_bars.astype(jnp.float32).reshape(1, TIMESTEPS)
    return pl.pallas_call(
        _body,
        out_shape=jax.ShapeDtypeStruct((ROWS, COLS), jnp.int32),
    )(t2, ab2, x_start)


# speedup vs baseline: 1.0433x; 1.0150x over previous
import numpy as np
import jax
import jax.numpy as jnp
from jax import lax
from jax.experimental import pallas as pl
from jax.experimental.pallas import tpu as pltpu

VOCAB = 100000
ROWS, COLS = 128, 4096
TIMESTEPS = 1000
MOD31 = (1 << 31) % VOCAB

_ROTS = ((13, 15, 26, 6), (17, 29, 16, 24))


def _np_threefry(k0, k1, x0, x1):
    with np.errstate(over="ignore"):
        k0, k1 = np.uint32(k0), np.uint32(k1)
        x0, x1 = np.uint32(x0), np.uint32(x1)
        ks = (k0, k1, np.uint32(k0 ^ k1 ^ np.uint32(0x1BD11BDA)))
        x0 = x0 + ks[0]
        x1 = x1 + ks[1]
        for i in range(5):
            for r in _ROTS[i % 2]:
                x0 = x0 + x1
                x1 = (x1 << np.uint32(r)) | (x1 >> np.uint32(32 - r))
                x1 = x1 ^ x0
            x0 = x0 + ks[(i + 1) % 3]
            x1 = x1 + ks[(i + 2) % 3] + np.uint32(i + 1)
        return x0, x1


def _np_split(k):
    a0, b0 = _np_threefry(k[0], k[1], 0, 0)
    a1, b1 = _np_threefry(k[0], k[1], 0, 1)
    return (a0, b0), (a1, b1)


_KN, _KB = _np_split((np.uint32(0), np.uint32(42)))
_K1, _K2 = _np_split(_KN)


def _u32(v):
    return np.uint32(v)


def _tf_bits(k0, k1, x1_in):
    ks = (_u32(k0), _u32(k1), _u32(int(k0) ^ int(k1) ^ 0x1BD11BDA))
    x1 = x1_in + ks[1]
    x0 = x1_in + _u32((int(ks[0]) + int(ks[1])) & 0xFFFFFFFF)
    for i in range(5):
        for j, r in enumerate(_ROTS[i % 2]):
            if i or j:
                x0 = x0 + x1
            x1 = ((x1 << _u32(r)) | (x1 >> _u32(32 - r))) ^ x0
        x0 = x0 + ks[(i + 1) % 3]
        x1 = x1 + _u32((int(ks[(i + 2) % 3]) + i + 1) & 0xFFFFFFFF)
    return x0 ^ x1


def _umod_vocab(bits_u32):
    f = bits_u32.astype(jnp.float32)
    q = (f * np.float32((1.0 + 1e-6) / VOCAB)).astype(jnp.uint32)
    r = bits_u32 - q * _u32(VOCAB)
    return jnp.where(r >= _u32(0x80000000), r + _u32(VOCAB), r)


def _body(t_ref, ab_ref, x_ref, o_ref):
    t = t_ref[:]
    steps = lax.broadcasted_iota(jnp.int32, (ROWS, TIMESTEPS), 1)
    ab = ab_ref[:]
    a_row = jnp.sum(jnp.where(t == steps, ab, 0.0), axis=1, keepdims=True)
    thr = jnp.ceil(a_row * np.float32(1 << 23)).astype(jnp.uint32)

    row = lax.broadcasted_iota(jnp.uint32, (ROWS, COLS), 0)
    col = lax.broadcasted_iota(jnp.uint32, (ROWS, COLS), 1)
    idx = row * _u32(COLS) + col

    noise = _umod_vocab(_tf_bits(_K2[0], _K2[1], idx)).astype(jnp.int32)
    ubits = _tf_bits(_KB[0], _KB[1], idx)
    keep = (ubits >> _u32(9)) < thr
    o_ref[:] = jnp.where(keep, x_ref[:], noise)


@jax.jit
def kernel(x_start, t, alpha_bars):
    x_start = x_start.astype(jnp.int32)
    t2 = t.astype(jnp.int32).reshape(ROWS, 1)
    ab2 = alpha_bars.astype(jnp.float32).reshape(1, TIMESTEPS)
    return pl.pallas_call(
        _body,
        out_shape=jax.ShapeDtypeStruct((ROWS, COLS), jnp.int32),
    )(t2, ab2, x_start)
